# baseline (speedup 1.0000x reference)
import math

import jax
import jax.numpy as jnp
from jax import lax
from jax.experimental import pallas as pl
from jax.experimental.pallas import tpu as pltpu
from jax.experimental.pallas import tpu_sc as plsc

N = 10000
E = 320000
D = 128

NC = 2
NS = 16
NW = NC * NS

CHUNK = 128
NCHUNK = 2 * (-(-E // (NW * CHUNK * 2)))
NPAIR = NCHUNK // 2
EPT = NCHUNK * CHUNK
E_PAD = EPT * NW

N_PAD = 10112
ROWS_PER_TILE = N_PAD // NS

INV_SQRT = float(1.0 / math.sqrt(1.0 + 1e-5))

ROW_BLK = 2000


def _sc_body(y2_hbm, zeros_hbm, idx_hbm, out_hbm,
             idxb, buf, agg, semg0, semg1, semi0, semi1):
    c = lax.axis_index("c")
    s = lax.axis_index("s")
    w = s * NC + c

    pltpu.sync_copy(zeros_hbm.at[pl.ds(s * ROWS_PER_TILE, ROWS_PER_TILE)],
                    agg.at[pl.ds(s * ROWS_PER_TILE, ROWS_PER_TILE)])
    plsc.subcore_barrier()

    semg = (semg0, semg1)
    semi = (semi0, semi1)

    pltpu.sync_copy(idx_hbm.at[w, 0], idxb.at[0])
    pltpu.async_copy(idx_hbm.at[w, 1], idxb.at[1], semi1)

    def pairs(p2, _):
        for pb in range(2):
            p = p2 * 2 + pb

            @pl.when(p > 0)
            def _():
                pltpu.make_async_copy(idx_hbm.at[w, p], idxb.at[pb],
                                      semi[pb]).wait()

            d0 = pltpu.async_copy(y2_hbm.at[idxb.at[pb, 0, 0]], buf.at[0],
                                  semg[0])
            d1 = pltpu.async_copy(y2_hbm.at[idxb.at[pb, 1, 0]], buf.at[1],
                                  semg[1])
            d0.wait()
            d1.wait()
            pltpu.sync_copy(buf.at[0], agg.at[idxb.at[pb, 0, 1]], add=True)
            pltpu.sync_copy(buf.at[1], agg.at[idxb.at[pb, 1, 1]], add=True)

            @pl.when(p + 2 < NPAIR)
            def _():
                pltpu.async_copy(idx_hbm.at[w, p + 2], idxb.at[pb], semi[pb])
        return 0

    lax.fori_loop(0, NPAIR // 2, pairs, 0)

    plsc.subcore_barrier()
    pltpu.sync_copy(agg.at[pl.ds(s * ROWS_PER_TILE, ROWS_PER_TILE)],
                    out_hbm.at[c].at[pl.ds(s * ROWS_PER_TILE, ROWS_PER_TILE)])


_sc_agg = pl.kernel(
    _sc_body,
    out_type=jax.ShapeDtypeStruct((NC, N_PAD, D), jnp.float32),
    mesh=plsc.VectorSubcoreMesh(core_axis_name="c", subcore_axis_name="s"),
    scratch_types=[
        pltpu.VMEM((2, 2, 2, CHUNK), jnp.int32),
        pltpu.VMEM((2, CHUNK, D), jnp.float32),
        pltpu.VMEM_SHARED((N_PAD, D), jnp.float32),
        pltpu.SemaphoreType.DMA,
        pltpu.SemaphoreType.DMA,
        pltpu.SemaphoreType.DMA,
        pltpu.SemaphoreType.DMA,
    ],
)


def _mm2_body(x_ref, w1_ref, w2_ref, y1_ref, y2_ref):
    xb = x_ref[...]
    y1_ref[...] = jnp.dot(xb, w1_ref[...], preferred_element_type=jnp.float32)
    y2_ref[...] = jnp.dot(xb, w2_ref[...], preferred_element_type=jnp.float32)


def _tc_mm2(x, w1, w2):
    return pl.pallas_call(
        _mm2_body,
        grid=(N // ROW_BLK,),
        in_specs=[
            pl.BlockSpec((ROW_BLK, D), lambda i: (i, 0)),
            pl.BlockSpec((D, D), lambda i: (0, 0)),
            pl.BlockSpec((D, D), lambda i: (0, 0)),
        ],
        out_specs=[pl.BlockSpec((ROW_BLK, D), lambda i: (i, 0))] * 2,
        out_shape=[jax.ShapeDtypeStruct((N, D), jnp.float32)] * 2,
    )(x, w1, w2)


def _fuse_mm2_body(y1_ref, a0_ref, a1_ref, g_ref, b_ref, w1_ref, w2_ref,
                   o1_ref, o2_ref):
    h = (y1_ref[...] + a0_ref[...] + a1_ref[...]) * (g_ref[...] * INV_SQRT)
    h = jnp.maximum(h + b_ref[...], 0.0)
    o1_ref[...] = jnp.dot(h, w1_ref[...], preferred_element_type=jnp.float32)
    o2_ref[...] = jnp.dot(h, w2_ref[...], preferred_element_type=jnp.float32)


def _tc_fuse_mm2(y1, a0, a1, gamma, beta, w1, w2):
    return pl.pallas_call(
        _fuse_mm2_body,
        grid=(N // ROW_BLK,),
        in_specs=[
            pl.BlockSpec((ROW_BLK, D), lambda i: (i, 0)),
            pl.BlockSpec((ROW_BLK, D), lambda i: (i, 0)),
            pl.BlockSpec((ROW_BLK, D), lambda i: (i, 0)),
            pl.BlockSpec((1, D), lambda i: (0, 0)),
            pl.BlockSpec((1, D), lambda i: (0, 0)),
            pl.BlockSpec((D, D), lambda i: (0, 0)),
            pl.BlockSpec((D, D), lambda i: (0, 0)),
        ],
        out_specs=[pl.BlockSpec((ROW_BLK, D), lambda i: (i, 0))] * 2,
        out_shape=[jax.ShapeDtypeStruct((N, D), jnp.float32)] * 2,
    )(y1, a0, a1, gamma, beta, w1, w2)


def _final_body(y1_ref, a0_ref, a1_ref, g_ref, b_ref, o_ref):
    h = (y1_ref[...] + a0_ref[...] + a1_ref[...]) * (g_ref[...] * INV_SQRT)
    o_ref[...] = jnp.maximum(h + b_ref[...], 0.0)


def _tc_final(y1, a0, a1, gamma, beta):
    return pl.pallas_call(
        _final_body,
        grid=(N // ROW_BLK,),
        in_specs=[
            pl.BlockSpec((ROW_BLK, D), lambda i: (i, 0)),
            pl.BlockSpec((ROW_BLK, D), lambda i: (i, 0)),
            pl.BlockSpec((ROW_BLK, D), lambda i: (i, 0)),
            pl.BlockSpec((1, D), lambda i: (0, 0)),
            pl.BlockSpec((1, D), lambda i: (0, 0)),
        ],
        out_specs=pl.BlockSpec((ROW_BLK, D), lambda i: (i, 0)),
        out_shape=jax.ShapeDtypeStruct((N, D), jnp.float32),
    )(y1, a0, a1, gamma, beta)


@jax.jit
def kernel(x, local_edge_index, W1_0, W2_0, gamma0, beta0,
           W1_1, W2_1, gamma1, beta1):
    row = local_edge_index[0]
    col = local_edge_index[1]
    pad = E_PAD - E
    colp = jnp.concatenate(
        [col, jnp.zeros((pad,), jnp.int32)]).reshape(NW, NPAIR, 2, 1, CHUNK)
    rowp = jnp.concatenate(
        [row, jnp.full((pad,), N, jnp.int32)]).reshape(NW, NPAIR, 2, 1, CHUNK)
    packed = jnp.concatenate([colp, rowp], axis=3)
    zeros = jnp.zeros((N_PAD, D), jnp.float32)
    g0 = gamma0.reshape(1, D)
    b0 = beta0.reshape(1, D)
    g1 = gamma1.reshape(1, D)
    b1 = beta1.reshape(1, D)

    y1_0, y2_0 = _tc_mm2(x, W1_0, W2_0)
    aggp0 = _sc_agg(y2_0, zeros, packed)
    y1_1, y2_1 = _tc_fuse_mm2(y1_0, aggp0[0, :N], aggp0[1, :N],
                              g0, b0, W1_1, W2_1)
    aggp1 = _sc_agg(y2_1, zeros, packed)
    return _tc_final(y1_1, aggp1[0, :N], aggp1[1, :N], g1, b1)


# speedup vs baseline: 1.0254x; 1.0254x over previous
import math

import jax
import jax.numpy as jnp
from jax import lax
from jax.experimental import pallas as pl
from jax.experimental.pallas import tpu as pltpu
from jax.experimental.pallas import tpu_sc as plsc

N = 10000
E = 320000
D = 128

NC = 2
NS = 16
NW = NC * NS

CHUNK = 128
NCHUNK = 2 * (-(-E // (NW * CHUNK * 2)))
NPAIR = NCHUNK // 2
EPT = NCHUNK * CHUNK
E_PAD = EPT * NW

N_PAD = 10112
ROWS_PER_TILE = N_PAD // NS

INV_SQRT = float(1.0 / math.sqrt(1.0 + 1e-5))

ROW_BLK = 2000


def _sc_body(y2_hbm, zeros_hbm, idx_hbm, out_hbm,
             idx_v, colb, rowb, buf, agg, semg0, semg1):
    c = lax.axis_index("c")
    s = lax.axis_index("s")
    w = s * NC + c

    pltpu.sync_copy(idx_hbm.at[w], idx_v)
    pltpu.sync_copy(zeros_hbm.at[pl.ds(s * ROWS_PER_TILE, ROWS_PER_TILE)],
                    agg.at[pl.ds(s * ROWS_PER_TILE, ROWS_PER_TILE)])
    plsc.subcore_barrier()

    semg = (semg0, semg1)

    def unpack(p, pb):
        for k in range(2):
            for i in range(CHUNK // 16):
                v = idx_v[p * 2 + k, pl.ds(i * 16, 16)]
                colb[pb, k, pl.ds(i * 16, 16)] = lax.bitwise_and(v, 0xFFFF)
                rowb[pb, k, pl.ds(i * 16, 16)] = lax.shift_right_logical(v, 16)

    unpack(0, 0)

    def pairs(p2, _):
        for pb in range(2):
            p = p2 * 2 + pb

            @pl.when(p + 1 < NPAIR)
            def _():
                unpack(p + 1, 1 - pb)

            d0 = pltpu.async_copy(y2_hbm.at[colb.at[pb, 0]], buf.at[0],
                                  semg[0])
            d1 = pltpu.async_copy(y2_hbm.at[colb.at[pb, 1]], buf.at[1],
                                  semg[1])
            d0.wait()
            d1.wait()
            pltpu.sync_copy(buf.at[0], agg.at[rowb.at[pb, 0]], add=True)
            pltpu.sync_copy(buf.at[1], agg.at[rowb.at[pb, 1]], add=True)
        return 0

    lax.fori_loop(0, NPAIR // 2, pairs, 0)

    plsc.subcore_barrier()
    pltpu.sync_copy(agg.at[pl.ds(s * ROWS_PER_TILE, ROWS_PER_TILE)],
                    out_hbm.at[c].at[pl.ds(s * ROWS_PER_TILE, ROWS_PER_TILE)])


_sc_agg = pl.kernel(
    _sc_body,
    out_type=jax.ShapeDtypeStruct((NC, N_PAD, D), jnp.float32),
    mesh=plsc.VectorSubcoreMesh(core_axis_name="c", subcore_axis_name="s"),
    scratch_types=[
        pltpu.VMEM((NCHUNK, CHUNK), jnp.int32),
        pltpu.VMEM((2, 2, CHUNK), jnp.int32),
        pltpu.VMEM((2, 2, CHUNK), jnp.int32),
        pltpu.VMEM((2, CHUNK, D), jnp.float32),
        pltpu.VMEM_SHARED((N_PAD, D), jnp.float32),
        pltpu.SemaphoreType.DMA,
        pltpu.SemaphoreType.DMA,
    ],
)


def _mm2_body(x_ref, w1_ref, w2_ref, y1_ref, y2_ref):
    xb = x_ref[...]
    y1_ref[...] = jnp.dot(xb, w1_ref[...], preferred_element_type=jnp.float32)
    y2_ref[...] = jnp.dot(xb, w2_ref[...], preferred_element_type=jnp.float32)


def _tc_mm2(x, w1, w2):
    return pl.pallas_call(
        _mm2_body,
        grid=(N // ROW_BLK,),
        in_specs=[
            pl.BlockSpec((ROW_BLK, D), lambda i: (i, 0)),
            pl.BlockSpec((D, D), lambda i: (0, 0)),
            pl.BlockSpec((D, D), lambda i: (0, 0)),
        ],
        out_specs=[pl.BlockSpec((ROW_BLK, D), lambda i: (i, 0))] * 2,
        out_shape=[jax.ShapeDtypeStruct((N, D), jnp.float32)] * 2,
    )(x, w1, w2)


def _fuse_mm2_body(y1_ref, a0_ref, a1_ref, g_ref, b_ref, w1_ref, w2_ref,
                   o1_ref, o2_ref):
    h = (y1_ref[...] + a0_ref[...] + a1_ref[...]) * (g_ref[...] * INV_SQRT)
    h = jnp.maximum(h + b_ref[...], 0.0)
    o1_ref[...] = jnp.dot(h, w1_ref[...], preferred_element_type=jnp.float32)
    o2_ref[...] = jnp.dot(h, w2_ref[...], preferred_element_type=jnp.float32)


def _tc_fuse_mm2(y1, a0, a1, gamma, beta, w1, w2):
    return pl.pallas_call(
        _fuse_mm2_body,
        grid=(N // ROW_BLK,),
        in_specs=[
            pl.BlockSpec((ROW_BLK, D), lambda i: (i, 0)),
            pl.BlockSpec((ROW_BLK, D), lambda i: (i, 0)),
            pl.BlockSpec((ROW_BLK, D), lambda i: (i, 0)),
            pl.BlockSpec((1, D), lambda i: (0, 0)),
            pl.BlockSpec((1, D), lambda i: (0, 0)),
            pl.BlockSpec((D, D), lambda i: (0, 0)),
            pl.BlockSpec((D, D), lambda i: (0, 0)),
        ],
        out_specs=[pl.BlockSpec((ROW_BLK, D), lambda i: (i, 0))] * 2,
        out_shape=[jax.ShapeDtypeStruct((N, D), jnp.float32)] * 2,
    )(y1, a0, a1, gamma, beta, w1, w2)


def _final_body(y1_ref, a0_ref, a1_ref, g_ref, b_ref, o_ref):
    h = (y1_ref[...] + a0_ref[...] + a1_ref[...]) * (g_ref[...] * INV_SQRT)
    o_ref[...] = jnp.maximum(h + b_ref[...], 0.0)


def _tc_final(y1, a0, a1, gamma, beta):
    return pl.pallas_call(
        _final_body,
        grid=(N // ROW_BLK,),
        in_specs=[
            pl.BlockSpec((ROW_BLK, D), lambda i: (i, 0)),
            pl.BlockSpec((ROW_BLK, D), lambda i: (i, 0)),
            pl.BlockSpec((ROW_BLK, D), lambda i: (i, 0)),
            pl.BlockSpec((1, D), lambda i: (0, 0)),
            pl.BlockSpec((1, D), lambda i: (0, 0)),
        ],
        out_specs=pl.BlockSpec((ROW_BLK, D), lambda i: (i, 0)),
        out_shape=jax.ShapeDtypeStruct((N, D), jnp.float32),
    )(y1, a0, a1, gamma, beta)


@jax.jit
def kernel(x, local_edge_index, W1_0, W2_0, gamma0, beta0,
           W1_1, W2_1, gamma1, beta1):
    row = local_edge_index[0]
    col = local_edge_index[1]
    pad = E_PAD - E
    packed = jnp.concatenate(
        [jnp.left_shift(row, 16) + col,
         jnp.full((pad,), N << 16, jnp.int32)]).reshape(NW, NCHUNK, CHUNK)
    zeros = jnp.zeros((N_PAD, D), jnp.float32)
    g0 = gamma0.reshape(1, D)
    b0 = beta0.reshape(1, D)
    g1 = gamma1.reshape(1, D)
    b1 = beta1.reshape(1, D)

    y1_0, y2_0 = _tc_mm2(x, W1_0, W2_0)
    aggp0 = _sc_agg(y2_0, zeros, packed)
    y1_1, y2_1 = _tc_fuse_mm2(y1_0, aggp0[0, :N], aggp0[1, :N],
                              g0, b0, W1_1, W2_1)
    aggp1 = _sc_agg(y2_1, zeros, packed)
    return _tc_final(y1_1, aggp1[0, :N], aggp1[1, :N], g1, b1)
